# baseline (speedup 1.0000x reference)
import functools

import jax
import jax.numpy as jnp
from jax import lax
from jax.experimental import pallas as pl
from jax.experimental.pallas import tpu as pltpu
from jax.experimental.pallas import tpu_sc as plsc

MAX_Z = 86
FEAT = 128

_NC = 2
_NS = 16
_NW = _NC * _NS
_CH = 400
_NCH = 8
_BPW = _CH * _NCH
_BPAD = _NW * _BPW
_GRP = _CH // 16


def _fuse_body(ec_ref, mw_ref, zt_ref, out_ref):
    out_ref[...] = lax.dot_general(
        ec_ref[...], mw_ref[...], (((1,), (1,)), ((), ())),
        preferred_element_type=jnp.float32,
    ) + zt_ref[...]


def _build_fused_table(elec_config, m_weight, z_table):
    return pl.pallas_call(
        _fuse_body,
        out_shape=jax.ShapeDtypeStruct((MAX_Z, FEAT), jnp.float32),
    )(elec_config[:MAX_Z], m_weight, z_table)


def _sc_gather(table_flat, idx):
    mesh = plsc.VectorSubcoreMesh(core_axis_name="c", subcore_axis_name="s")

    @functools.partial(
        pl.kernel,
        mesh=mesh,
        out_type=jax.ShapeDtypeStruct((_BPAD, FEAT), jnp.float32),
        compiler_params=pltpu.CompilerParams(needs_layout_passes=False),
        scratch_types=[
            pltpu.VMEM((MAX_Z * FEAT,), jnp.float32),
            pltpu.VMEM((_BPW,), jnp.int32),
            pltpu.VMEM((_CH, FEAT), jnp.float32),
            pltpu.VMEM((_CH, FEAT), jnp.float32),
            pltpu.SemaphoreType.DMA,
            pltpu.SemaphoreType.DMA,
        ],
    )
    def k(table_hbm, idx_hbm, out_hbm, table_v, idx_v, buf_a, buf_b,
          ssem_a, ssem_b):
        wid = lax.axis_index("s") * _NC + lax.axis_index("c")
        base = pl.multiple_of(wid * _BPW, 8)
        pltpu.sync_copy(table_hbm, table_v)
        pltpu.sync_copy(idx_hbm.at[pl.ds(base, _BPW)], idx_v)

        iota = lax.iota(jnp.int32, 16)

        def compute_chunk(j, buf):
            def grp(g, _):
                zpos = j * _CH + g * 16
                z16 = plsc.load_gather(idx_v, [iota + zpos])
                ridx = iota + g * 16

                def col(c, carry):
                    gidx, cidx = carry
                    vals = plsc.load_gather(table_v, [gidx])
                    plsc.store_scatter(buf, [ridx, cidx], vals)
                    return gidx + 1, cidx + 1

                lax.fori_loop(0, FEAT, col,
                              (z16 * FEAT, jnp.zeros((16,), jnp.int32)),
                              unroll=16)
                return _

            lax.fori_loop(0, _GRP, grp, 0)

        def store(j, buf, sem):
            return pltpu.make_async_copy(
                buf, out_hbm.at[pl.ds(base + j * _CH, _CH)], sem)

        compute_chunk(0, buf_a)

        def body(i, carry):
            j0 = 2 * i
            store(j0, buf_a, ssem_a).start()
            compute_chunk(j0 + 1, buf_b)
            store(j0 + 1, buf_b, ssem_b).start()
            store(j0, buf_a, ssem_a).wait()
            compute_chunk(j0 + 2, buf_a)
            store(j0 + 1, buf_b, ssem_b).wait()
            return carry

        lax.fori_loop(0, (_NCH - 2) // 2, body, 0)

        last = _NCH - 1
        store(last - 1, buf_a, ssem_a).start()
        compute_chunk(last, buf_b)
        store(last - 1, buf_a, ssem_a).wait()
        store(last, buf_b, ssem_b).start()
        store(last, buf_b, ssem_b).wait()

    return k(table_flat, idx)


def kernel(z, elec_config, m_weight, z_table):
    fused = _build_fused_table(elec_config, m_weight, z_table)
    zi = z.astype(jnp.int32)
    n = zi.shape[0]
    zi_pad = jnp.zeros((_BPAD,), jnp.int32).at[:n].set(zi)
    out = _sc_gather(fused.reshape(MAX_Z * FEAT), zi_pad)
    return out[:n]


# speedup vs baseline: 3.0359x; 3.0359x over previous
import functools

import jax
import jax.numpy as jnp
from jax import lax
from jax.experimental import pallas as pl
from jax.experimental.pallas import tpu as pltpu
from jax.experimental.pallas import tpu_sc as plsc

MAX_Z = 86
FEAT = 128
N_ROWS = 100000

_NC = 2
_NS = 16
_NW = _NC * _NS
_CH = 128
_BPW = 3128
_LAST_BPW = N_ROWS - (_NW - 1) * _BPW
_NFULL = _BPW // _CH
_TAIL = _BPW - _NFULL * _CH
_LNFULL = _LAST_BPW // _CH
_LTAIL = _LAST_BPW - _LNFULL * _CH


def _fuse_body(ec_ref, mw_ref, zt_ref, out_ref):
    out_ref[...] = lax.dot_general(
        ec_ref[...], mw_ref[...], (((1,), (1,)), ((), ())),
        preferred_element_type=jnp.float32,
    ) + zt_ref[...]


def _build_fused_table(elec_config, m_weight, z_table):
    return pl.pallas_call(
        _fuse_body,
        out_shape=jax.ShapeDtypeStruct((MAX_Z, FEAT), jnp.float32),
    )(elec_config[:MAX_Z], m_weight, z_table)


def _sc_gather(table, idx):
    mesh = plsc.VectorSubcoreMesh(core_axis_name="c", subcore_axis_name="s")

    @functools.partial(
        pl.kernel,
        mesh=mesh,
        out_type=jax.ShapeDtypeStruct((N_ROWS, FEAT), jnp.float32),
        scratch_types=[
            pltpu.VMEM((_BPW,), jnp.int32),
            pltpu.VMEM((_CH, FEAT), jnp.float32),
            pltpu.VMEM((_CH, FEAT), jnp.float32),
            pltpu.SemaphoreType.DMA,
            pltpu.SemaphoreType.DMA,
            pltpu.SemaphoreType.DMA,
            pltpu.SemaphoreType.DMA,
        ],
    )
    def k(table_hbm, idx_hbm, out_hbm, idx_v, buf_a, buf_b,
          gsem_a, gsem_b, ssem_a, ssem_b):
        wid = lax.axis_index("s") * _NC + lax.axis_index("c")
        base = pl.multiple_of(wid * _BPW, 8)

        def gather(j, rows, buf, sem):
            return pltpu.make_async_copy(
                table_hbm.at[idx_v.at[pl.ds(j * _CH, rows)]],
                buf.at[pl.ds(0, rows)], sem)

        def store(j, rows, buf, sem):
            return pltpu.make_async_copy(
                buf.at[pl.ds(0, rows)],
                out_hbm.at[pl.ds(base + j * _CH, rows)], sem)

        @pl.when(wid < _NW - 1)
        def _full_worker():
            pltpu.sync_copy(idx_hbm.at[pl.ds(base, _BPW)], idx_v)
            gather(0, _CH, buf_a, gsem_a).start()

            def body(i, carry):
                j0 = 2 * i
                gather(j0, _CH, buf_a, gsem_a).wait()

                @pl.when(i > 0)
                def _():
                    store(j0 - 1, _CH, buf_b, ssem_b).wait()

                gather(j0 + 1, _CH, buf_b, gsem_b).start()
                store(j0, _CH, buf_a, ssem_a).start()
                gather(j0 + 1, _CH, buf_b, gsem_b).wait()
                store(j0, _CH, buf_a, ssem_a).wait()
                gather(j0 + 2, _CH, buf_a, gsem_a).start()
                store(j0 + 1, _CH, buf_b, ssem_b).start()
                return carry

            lax.fori_loop(0, (_NFULL - 2) // 2, body, 0)

            p, q = _NFULL - 2, _NFULL - 1
            gather(p, _CH, buf_a, gsem_a).wait()
            store(p - 1, _CH, buf_b, ssem_b).wait()
            gather(q, _CH, buf_b, gsem_b).start()
            store(p, _CH, buf_a, ssem_a).start()
            gather(q, _CH, buf_b, gsem_b).wait()
            store(p, _CH, buf_a, ssem_a).wait()
            gather(_NFULL, _TAIL, buf_a, gsem_a).start()
            store(q, _CH, buf_b, ssem_b).start()
            gather(_NFULL, _TAIL, buf_a, gsem_a).wait()
            store(_NFULL, _TAIL, buf_a, ssem_a).start()
            store(q, _CH, buf_b, ssem_b).wait()
            store(_NFULL, _TAIL, buf_a, ssem_a).wait()

        @pl.when(wid == _NW - 1)
        def _last_worker():
            pltpu.sync_copy(idx_hbm.at[pl.ds(base, _LAST_BPW)],
                            idx_v.at[pl.ds(0, _LAST_BPW)])

            def body(j, carry):
                gather(j, _CH, buf_a, gsem_a).start()
                gather(j, _CH, buf_a, gsem_a).wait()
                store(j, _CH, buf_a, ssem_a).start()
                store(j, _CH, buf_a, ssem_a).wait()
                return carry

            lax.fori_loop(0, _LNFULL, body, 0)
            gather(_LNFULL, _LTAIL, buf_a, gsem_a).start()
            gather(_LNFULL, _LTAIL, buf_a, gsem_a).wait()
            store(_LNFULL, _LTAIL, buf_a, ssem_a).start()
            store(_LNFULL, _LTAIL, buf_a, ssem_a).wait()

    return k(table, idx)


def kernel(z, elec_config, m_weight, z_table):
    fused = _build_fused_table(elec_config, m_weight, z_table)
    return _sc_gather(fused, z.astype(jnp.int32))


# speedup vs baseline: 9.3651x; 3.0847x over previous
import functools

import jax
import jax.numpy as jnp
from jax import lax
from jax.experimental import pallas as pl
from jax.experimental.pallas import tpu as pltpu
from jax.experimental.pallas import tpu_sc as plsc

MAX_Z = 86
FEAT = 128
N_ROWS = 100000

_NC = 2
_NS = 16
_NW = _NC * _NS
_CH = 128
_BPW = 3128
_LAST_BPW = N_ROWS - (_NW - 1) * _BPW
_NFULL = _BPW // _CH
_TAIL = _BPW - _NFULL * _CH
_LNFULL = _LAST_BPW // _CH
_LTAIL = _LAST_BPW - _LNFULL * _CH


def _fuse_body(ec_ref, mw_ref, zt_ref, out_ref):
    out_ref[...] = lax.dot_general(
        ec_ref[...], mw_ref[...], (((1,), (1,)), ((), ())),
        preferred_element_type=jnp.float32,
    ) + zt_ref[...]


def _build_fused_table(elec_config, m_weight, z_table):
    return pl.pallas_call(
        _fuse_body,
        out_shape=jax.ShapeDtypeStruct((MAX_Z, FEAT), jnp.float32),
    )(elec_config[:MAX_Z], m_weight, z_table)


def _sc_gather(table, idx):
    mesh = plsc.VectorSubcoreMesh(core_axis_name="c", subcore_axis_name="s")

    @functools.partial(
        pl.kernel,
        mesh=mesh,
        out_type=jax.ShapeDtypeStruct((N_ROWS, FEAT), jnp.float32),
        scratch_types=[
            pltpu.VMEM((_BPW,), jnp.int32),
            pltpu.VMEM((_CH, FEAT), jnp.float32),
            pltpu.VMEM((_CH, FEAT), jnp.float32),
            pltpu.VMEM_SHARED((MAX_Z, FEAT), jnp.float32),
            pltpu.SemaphoreType.DMA,
            pltpu.SemaphoreType.DMA,
            pltpu.SemaphoreType.DMA,
            pltpu.SemaphoreType.DMA,
        ],
    )
    def k(table_hbm, idx_hbm, out_hbm, idx_v, buf_a, buf_b, table_spm,
          gsem_a, gsem_b, ssem_a, ssem_b):
        wid = lax.axis_index("s") * _NC + lax.axis_index("c")
        base = pl.multiple_of(wid * _BPW, 8)

        @pl.when(lax.axis_index("s") == 0)
        def _():
            pltpu.sync_copy(table_hbm, table_spm)

        plsc.subcore_barrier()

        def gather(j, rows, buf, sem):
            return pltpu.make_async_copy(
                table_spm.at[idx_v.at[pl.ds(j * _CH, rows)]],
                buf.at[pl.ds(0, rows)], sem)

        def store(j, rows, buf, sem):
            return pltpu.make_async_copy(
                buf.at[pl.ds(0, rows)],
                out_hbm.at[pl.ds(base + j * _CH, rows)], sem)

        @pl.when(wid < _NW - 1)
        def _full_worker():
            pltpu.sync_copy(idx_hbm.at[pl.ds(base, _BPW)], idx_v)
            gather(0, _CH, buf_a, gsem_a).start()

            def body(i, carry):
                j0 = 2 * i
                gather(j0, _CH, buf_a, gsem_a).wait()

                @pl.when(i > 0)
                def _():
                    store(j0 - 1, _CH, buf_b, ssem_b).wait()

                gather(j0 + 1, _CH, buf_b, gsem_b).start()
                store(j0, _CH, buf_a, ssem_a).start()
                gather(j0 + 1, _CH, buf_b, gsem_b).wait()
                store(j0, _CH, buf_a, ssem_a).wait()
                gather(j0 + 2, _CH, buf_a, gsem_a).start()
                store(j0 + 1, _CH, buf_b, ssem_b).start()
                return carry

            lax.fori_loop(0, (_NFULL - 2) // 2, body, 0)

            p, q = _NFULL - 2, _NFULL - 1
            gather(p, _CH, buf_a, gsem_a).wait()
            store(p - 1, _CH, buf_b, ssem_b).wait()
            gather(q, _CH, buf_b, gsem_b).start()
            store(p, _CH, buf_a, ssem_a).start()
            gather(q, _CH, buf_b, gsem_b).wait()
            store(p, _CH, buf_a, ssem_a).wait()
            gather(_NFULL, _TAIL, buf_a, gsem_a).start()
            store(q, _CH, buf_b, ssem_b).start()
            gather(_NFULL, _TAIL, buf_a, gsem_a).wait()
            store(_NFULL, _TAIL, buf_a, ssem_a).start()
            store(q, _CH, buf_b, ssem_b).wait()
            store(_NFULL, _TAIL, buf_a, ssem_a).wait()

        @pl.when(wid == _NW - 1)
        def _last_worker():
            pltpu.sync_copy(idx_hbm.at[pl.ds(base, _LAST_BPW)],
                            idx_v.at[pl.ds(0, _LAST_BPW)])

            def body(j, carry):
                gather(j, _CH, buf_a, gsem_a).start()
                gather(j, _CH, buf_a, gsem_a).wait()
                store(j, _CH, buf_a, ssem_a).start()
                store(j, _CH, buf_a, ssem_a).wait()
                return carry

            lax.fori_loop(0, _LNFULL, body, 0)
            gather(_LNFULL, _LTAIL, buf_a, gsem_a).start()
            gather(_LNFULL, _LTAIL, buf_a, gsem_a).wait()
            store(_LNFULL, _LTAIL, buf_a, ssem_a).start()
            store(_LNFULL, _LTAIL, buf_a, ssem_a).wait()

    return k(table, idx)


def kernel(z, elec_config, m_weight, z_table):
    fused = _build_fused_table(elec_config, m_weight, z_table)
    return _sc_gather(fused, z.astype(jnp.int32))


# speedup vs baseline: 9.7625x; 1.0424x over previous
import functools

import jax
import jax.numpy as jnp
from jax import lax
from jax.experimental import pallas as pl
from jax.experimental.pallas import tpu as pltpu
from jax.experimental.pallas import tpu_sc as plsc

MAX_Z = 86
FEAT = 128
N_ROWS = 100000

_NC = 2
_NS = 16
_NW = _NC * _NS
_CH = 128
_BPW = 3128
_LAST_BPW = N_ROWS - (_NW - 1) * _BPW
_NFULL = _BPW // _CH
_TAIL = _BPW - _NFULL * _CH
_LNFULL = _LAST_BPW // _CH
_LTAIL = _LAST_BPW - _LNFULL * _CH


def _fuse_body(ec_ref, mw_ref, zt_ref, out_ref):
    out_ref[...] = lax.dot_general(
        ec_ref[...], mw_ref[...], (((1,), (1,)), ((), ())),
        preferred_element_type=jnp.float32,
    ) + zt_ref[...]


def _build_fused_table(elec_config, m_weight, z_table):
    return pl.pallas_call(
        _fuse_body,
        out_shape=jax.ShapeDtypeStruct((MAX_Z, FEAT), jnp.float32),
    )(elec_config[:MAX_Z], m_weight, z_table)


def _sc_gather(table, idx):
    mesh = plsc.VectorSubcoreMesh(core_axis_name="c", subcore_axis_name="s")

    @functools.partial(
        pl.kernel,
        mesh=mesh,
        out_type=jax.ShapeDtypeStruct((N_ROWS, FEAT), jnp.float32),
        scratch_types=[
            pltpu.VMEM((_BPW,), jnp.int32),
            pltpu.VMEM((_CH, FEAT), jnp.float32),
            pltpu.VMEM((_CH, FEAT), jnp.float32),
            pltpu.VMEM_SHARED((MAX_Z, FEAT), jnp.float32),
            pltpu.SemaphoreType.DMA,
            pltpu.SemaphoreType.DMA,
            pltpu.SemaphoreType.DMA,
            pltpu.SemaphoreType.DMA,
        ],
    )
    def k(table_hbm, idx_hbm, out_hbm, idx_v, buf_a, buf_b, table_spm,
          gsem_a, gsem_b, ssem_a, ssem_b):
        wid = lax.axis_index("s") * _NC + lax.axis_index("c")
        base = pl.multiple_of(wid * _BPW, 8)

        @pl.when(lax.axis_index("s") == 0)
        def _():
            pltpu.sync_copy(table_hbm, table_spm)

        plsc.subcore_barrier()

        def gather(j, rows, buf, sem):
            return pltpu.make_async_copy(
                table_spm.at[idx_v.at[pl.ds(j * _CH, rows)]],
                buf.at[pl.ds(0, rows)], sem)

        def store(j, rows, buf, sem):
            return pltpu.make_async_copy(
                buf.at[pl.ds(0, rows)],
                out_hbm.at[pl.ds(base + j * _CH, rows)], sem)

        @pl.when(wid < _NW - 1)
        def _full_worker():
            pltpu.sync_copy(idx_hbm.at[pl.ds(base, _BPW)], idx_v)
            gather(0, _CH, buf_a, gsem_a).start()

            def body(i, carry):
                j0 = 2 * i
                gather(j0, _CH, buf_a, gsem_a).wait()

                @pl.when(i > 0)
                def _():
                    store(j0 - 1, _CH, buf_b, ssem_b).wait()

                gather(j0 + 1, _CH, buf_b, gsem_b).start()
                store(j0, _CH, buf_a, ssem_a).start()
                gather(j0 + 1, _CH, buf_b, gsem_b).wait()
                store(j0, _CH, buf_a, ssem_a).wait()
                gather(j0 + 2, _CH, buf_a, gsem_a).start()
                store(j0 + 1, _CH, buf_b, ssem_b).start()
                return carry

            lax.fori_loop(0, (_NFULL - 2) // 2, body, 0)

            p, q = _NFULL - 2, _NFULL - 1
            gather(p, _CH, buf_a, gsem_a).wait()
            store(p - 1, _CH, buf_b, ssem_b).wait()
            gather(q, _CH, buf_b, gsem_b).start()
            store(p, _CH, buf_a, ssem_a).start()
            gather(q, _CH, buf_b, gsem_b).wait()
            store(p, _CH, buf_a, ssem_a).wait()
            gather(_NFULL, _TAIL, buf_a, gsem_a).start()
            store(q, _CH, buf_b, ssem_b).start()
            gather(_NFULL, _TAIL, buf_a, gsem_a).wait()
            store(_NFULL, _TAIL, buf_a, ssem_a).start()
            store(q, _CH, buf_b, ssem_b).wait()
            store(_NFULL, _TAIL, buf_a, ssem_a).wait()

        @pl.when(wid == _NW - 1)
        def _last_worker():
            pltpu.sync_copy(idx_hbm.at[pl.ds(base, _LAST_BPW)],
                            idx_v.at[pl.ds(0, _LAST_BPW)])

            def body(j, carry):
                gather(j, _CH, buf_a, gsem_a).start()
                gather(j, _CH, buf_a, gsem_a).wait()
                store(j, _CH, buf_a, ssem_a).start()
                store(j, _CH, buf_a, ssem_a).wait()
                return carry

            lax.fori_loop(0, _LNFULL, body, 0)
            gather(_LNFULL, _LTAIL, buf_a, gsem_a).start()
            gather(_LNFULL, _LTAIL, buf_a, gsem_a).wait()
            store(_LNFULL, _LTAIL, buf_a, ssem_a).start()
            store(_LNFULL, _LTAIL, buf_a, ssem_a).wait()

    return k(table, idx)


def kernel(z, elec_config, m_weight, z_table):
    return _sc_gather(z_table, z.astype(jnp.int32))
